# baseline (speedup 1.0000x reference)
import functools

import jax
import jax.numpy as jnp
from jax import lax
from jax.experimental import pallas as pl
from jax.experimental.pallas import tpu as pltpu, tpu_sc as plsc

NUM_CORES = 2
NUM_SUBCORES = 16
LANES = 16
NW = NUM_CORES * NUM_SUBCORES

BATCH = 16384
EMB_K = 16
NCLS = 5
BPW = BATCH // NW
CHUNK = 128
NCHUNK = BPW // CHUNK
GROUPS_PER_BLOCK = 4
ROWS_PER_BLOCK = GROUPS_PER_BLOCK * LANES
NBLOCKS = BPW // ROWS_PER_BLOCK


def _sc_body(x_hbm, W_hbm, H_hbm, lw_hbm, lb_hbm, out_hbm,
             xv, uidx_v, vidx_v, urows, vrows, outv, wv, bv, wubv, sem):
    wid = lax.axis_index("s") * NUM_CORES + lax.axis_index("c")
    base = wid * BPW

    pltpu.sync_copy(x_hbm.at[pl.ds(base, BPW)], xv)
    pltpu.sync_copy(lw_hbm, wv)
    pltpu.sync_copy(lb_hbm, bv.at[pl.ds(0, NCLS)])

    iota = lax.iota(jnp.int32, LANES)
    zerov = jnp.zeros((LANES,), jnp.int32)
    onev = jnp.full((LANES,), 1, jnp.int32)

    for c in range(NCHUNK):
        for o in range(CHUNK // LANES):
            rows = c * CHUNK + o * LANES + iota
            u = plsc.load_gather(xv, [rows, zerov])
            v = plsc.load_gather(xv, [rows, onev])
            uidx_v[c, pl.ds(o * LANES, LANES)] = u
            vidx_v[c, pl.ds(o * LANES, LANES)] = v

    copies = []
    for c in range(NCHUNK):
        copies.append(pltpu.async_copy(
            W_hbm.at[uidx_v.at[c]], urows.at[pl.ds(c * CHUNK, CHUNK)], sem))
        copies.append(pltpu.async_copy(
            H_hbm.at[vidx_v.at[c]], vrows.at[pl.ds(c * CHUNK, CHUNK)], sem))

    for j in range(NCLS):
        for h in range(2):
            wrow = wv[j, pl.ds(h * LANES, LANES)]
            for t in range(LANES):
                wubv[j, h * LANES + t] = jnp.broadcast_to(wrow[t], (LANES,))
    brow = bv[...]
    bvecs = [jnp.broadcast_to(brow[j], (LANES,)) for j in range(NCLS)]

    for cp in copies:
        cp.wait()

    def block(gb, carry):
        row0 = gb * ROWS_PER_BLOCK
        row_idx = [row0 + g * LANES + iota for g in range(GROUPS_PER_BLOCK)]
        acc = [[bvecs[j] for j in range(NCLS)]
               for _ in range(GROUPS_PER_BLOCK)]
        for k in range(2 * EMB_K):
            src = urows if k < EMB_K else vrows
            colv = jnp.full((LANES,), k % EMB_K, jnp.int32)
            wvecs = [wubv[j, k] for j in range(NCLS)]
            for g in range(GROUPS_PER_BLOCK):
                z = plsc.load_gather(src, [row_idx[g], colv])
                for j in range(NCLS):
                    acc[g][j] = acc[g][j] + z * wvecs[j]
        for g in range(GROUPS_PER_BLOCK):
            h = acc[g]
            m = h[0]
            for j in range(1, NCLS):
                m = jnp.maximum(m, h[j])
            e = [jnp.exp(h[j] - m) for j in range(NCLS)]
            s = e[0]
            for j in range(1, NCLS):
                s = s + e[j]
            r = jnp.full((LANES,), 1.0, jnp.float32) / s
            for j in range(NCLS):
                plsc.store_scatter(
                    outv, [row_idx[g], jnp.full((LANES,), j, jnp.int32)],
                    e[j] * r)
        return carry

    lax.fori_loop(0, NBLOCKS, block, 0)

    pltpu.sync_copy(outv, out_hbm.at[pl.ds(base, BPW)])


@jax.jit
def _run(x, W, H, lin_w, lin_b):
    mesh = plsc.VectorSubcoreMesh(
        core_axis_name="c", subcore_axis_name="s",
        num_cores=NUM_CORES, num_subcores=NUM_SUBCORES)
    return pl.kernel(
        _sc_body,
        out_type=jax.ShapeDtypeStruct((BATCH, NCLS), jnp.float32),
        mesh=mesh,
        compiler_params=pltpu.CompilerParams(
            needs_layout_passes=False, use_tc_tiling_on_sc=False),
        scratch_types=[
            pltpu.VMEM((BPW, 2), jnp.int32),
            pltpu.VMEM((NCHUNK, CHUNK), jnp.int32),
            pltpu.VMEM((NCHUNK, CHUNK), jnp.int32),
            pltpu.VMEM((BPW, EMB_K), jnp.float32),
            pltpu.VMEM((BPW, EMB_K), jnp.float32),
            pltpu.VMEM((BPW, NCLS), jnp.float32),
            pltpu.VMEM((NCLS, 2 * EMB_K), jnp.float32),
            pltpu.VMEM((LANES,), jnp.float32),
            pltpu.VMEM((NCLS, 2 * EMB_K, LANES), jnp.float32),
            pltpu.SemaphoreType.DMA,
        ],
    )(x, W, H, lin_w, lin_b)


def kernel(x, W, H, lin_w, lin_b):
    return _run(x, W, H, lin_w, lin_b)


# speedup vs baseline: 1.4532x; 1.4532x over previous
import jax
import jax.numpy as jnp
from jax import lax
from jax.experimental import pallas as pl
from jax.experimental.pallas import tpu as pltpu, tpu_sc as plsc

NUM_CORES = 2
NUM_SUBCORES = 16
LANES = 16
NW = NUM_CORES * NUM_SUBCORES

BATCH = 16384
EMB_K = 16
NCLS = 5
PJ = 16
NROWS = 1000000
NROWS_PAD = 1 << 20
BPW = BATCH // NW
CHUNK = 128
NCHUNK = BPW // CHUNK
GROUPS_PER_BLOCK = 4
ROWS_PER_BLOCK = GROUPS_PER_BLOCK * LANES
NBLOCKS = BPW // ROWS_PER_BLOCK

K1_COLS = 16384
K1_GRID = -(-NROWS // K1_COLS)
K1_NSUB = 128 // PJ
K1_SUB = K1_COLS // K1_NSUB
K1_OUT_ROWS = K1_SUB
NROWS_PACK = K1_GRID * K1_COLS


def _tc_body(wt_ref, ht_ref, wu_ref, wv_ref, bh_ref, pu_ref, pv_ref):
    bh = bh_ref[...]
    dn = (((0,), (0,)), ((), ()))
    for m in range(K1_NSUB):
        sl = pl.ds(m * K1_SUB, K1_SUB)
        ol = pl.ds(m * PJ, PJ)
        yu = lax.dot_general(wt_ref[:, sl], wu_ref[...], dn,
                             preferred_element_type=jnp.float32) + bh
        pu_ref[:, ol] = yu
        yv = lax.dot_general(ht_ref[:, sl], wv_ref[...], dn,
                             preferred_element_type=jnp.float32) + bh
        pv_ref[:, ol] = yv


def _make_tables(WT, HT, wu, wv, bh):
    out_shape = jax.ShapeDtypeStruct((NROWS_PACK * PJ // 128, 128),
                                     jnp.float32)
    return pl.pallas_call(
        _tc_body,
        grid=(K1_GRID,),
        in_specs=[
            pl.BlockSpec((EMB_K, K1_COLS), lambda g: (0, g)),
            pl.BlockSpec((EMB_K, K1_COLS), lambda g: (0, g)),
            pl.BlockSpec((EMB_K, PJ), lambda g: (0, 0)),
            pl.BlockSpec((EMB_K, PJ), lambda g: (0, 0)),
            pl.BlockSpec((1, PJ), lambda g: (0, 0)),
        ],
        out_specs=[
            pl.BlockSpec((K1_OUT_ROWS, 128), lambda g: (g, 0)),
            pl.BlockSpec((K1_OUT_ROWS, 128), lambda g: (g, 0)),
        ],
        out_shape=[out_shape, out_shape],
    )(WT, HT, wu, wv, bh)


def _sc_body(x_hbm, pu_hbm, pv_hbm, out_hbm,
             xv, uidx_v, vidx_v, purows, pvrows, outv, sem):
    wid = lax.axis_index("s") * NUM_CORES + lax.axis_index("c")
    base = wid * BPW

    pltpu.sync_copy(x_hbm.at[pl.ds(base, BPW)], xv)

    iota = lax.iota(jnp.int32, LANES)
    zerov = jnp.zeros((LANES,), jnp.int32)
    onev = jnp.full((LANES,), 1, jnp.int32)
    m_hi = jnp.full((LANES,), -K1_COLS, jnp.int32)
    m_lo = jnp.full((LANES,), K1_SUB - 1, jnp.int32)
    m_mid = jnp.full((LANES,), K1_NSUB - 1, jnp.int32)

    def packed(n):
        return ((n & m_hi) + ((n & m_lo) << 3)) + ((n >> 11) & m_mid)

    for c in range(NCHUNK):
        for o in range(CHUNK // LANES):
            rows = c * CHUNK + o * LANES + iota
            u = plsc.load_gather(xv, [rows, zerov])
            v = plsc.load_gather(xv, [rows, onev])
            uidx_v[c, pl.ds(o * LANES, LANES)] = packed(u)
            vidx_v[c, pl.ds(o * LANES, LANES)] = packed(v)

    copies = []
    for c in range(NCHUNK):
        copies.append(pltpu.async_copy(
            pu_hbm.at[uidx_v.at[c]], purows.at[pl.ds(c * CHUNK, CHUNK)], sem))
        copies.append(pltpu.async_copy(
            pv_hbm.at[vidx_v.at[c]], pvrows.at[pl.ds(c * CHUNK, CHUNK)], sem))
    for cp in copies:
        cp.wait()

    def block(gb, carry):
        row0 = gb * ROWS_PER_BLOCK
        for g in range(GROUPS_PER_BLOCK):
            row_idx = row0 + g * LANES + iota
            h = []
            for j in range(NCLS):
                colv = jnp.full((LANES,), j, jnp.int32)
                pu = plsc.load_gather(purows, [row_idx, colv])
                pv = plsc.load_gather(pvrows, [row_idx, colv])
                h.append(pu + pv)
            m = h[0]
            for j in range(1, NCLS):
                m = jnp.maximum(m, h[j])
            e = [jnp.exp(h[j] - m) for j in range(NCLS)]
            s = e[0]
            for j in range(1, NCLS):
                s = s + e[j]
            r = jnp.full((LANES,), 1.0, jnp.float32) / s
            for j in range(NCLS):
                plsc.store_scatter(
                    outv, [row_idx, jnp.full((LANES,), j, jnp.int32)],
                    e[j] * r)
        return carry

    lax.fori_loop(0, NBLOCKS, block, 0)

    pltpu.sync_copy(outv, out_hbm.at[pl.ds(base, BPW)])


def _score(x, pu, pv):
    mesh = plsc.VectorSubcoreMesh(
        core_axis_name="c", subcore_axis_name="s",
        num_cores=NUM_CORES, num_subcores=NUM_SUBCORES)
    return pl.kernel(
        _sc_body,
        out_type=jax.ShapeDtypeStruct((BATCH, NCLS), jnp.float32),
        mesh=mesh,
        compiler_params=pltpu.CompilerParams(
            needs_layout_passes=False, use_tc_tiling_on_sc=False),
        scratch_types=[
            pltpu.VMEM((BPW, 2), jnp.int32),
            pltpu.VMEM((NCHUNK, CHUNK), jnp.int32),
            pltpu.VMEM((NCHUNK, CHUNK), jnp.int32),
            pltpu.VMEM((BPW, PJ), jnp.float32),
            pltpu.VMEM((BPW, PJ), jnp.float32),
            pltpu.VMEM((BPW, NCLS), jnp.float32),
            pltpu.SemaphoreType.DMA,
        ],
    )(x, pu, pv)


@jax.jit
def _run(x, W, H, lin_w, lin_b):
    wu = jnp.zeros((EMB_K, PJ), jnp.float32).at[:, :NCLS].set(
        lin_w[:, :EMB_K].T)
    wv = jnp.zeros((EMB_K, PJ), jnp.float32).at[:, :NCLS].set(
        lin_w[:, EMB_K:].T)
    bh = jnp.zeros((1, PJ), jnp.float32).at[0, :NCLS].set(0.5 * lin_b)
    p2u, p2v = _make_tables(W.T, H.T, wu, wv, bh)
    pu = p2u.reshape(NROWS_PACK, PJ)
    pv = p2v.reshape(NROWS_PACK, PJ)
    return _score(x, pu, pv)


def kernel(x, W, H, lin_w, lin_b):
    return _run(x, W, H, lin_w, lin_b)


# speedup vs baseline: 2.2404x; 1.5417x over previous
import jax
import jax.numpy as jnp
from jax import lax
from jax.experimental import pallas as pl
from jax.experimental.pallas import tpu as pltpu, tpu_sc as plsc

NUM_CORES = 2
NUM_SUBCORES = 16
LANES = 16
NW = NUM_CORES * NUM_SUBCORES

BATCH = 16384
EMB_K = 16
NCLS = 5
PJ = 16
NROWS = 1000000
NROWS_PAD = 1 << 20
BPW = BATCH // NW
CHUNK = 128
NCHUNK = BPW // CHUNK
GROUPS_PER_BLOCK = 4
ROWS_PER_BLOCK = GROUPS_PER_BLOCK * LANES
NBLOCKS = BPW // ROWS_PER_BLOCK

K1_COLS = 16384
K1_GRID = -(-NROWS // K1_COLS)
K1_NSUB = 128 // PJ
K1_SUB = K1_COLS // K1_NSUB
K1_OUT_ROWS = K1_SUB
NROWS_PACK = K1_GRID * K1_COLS


def _tc_body(wt_ref, ht_ref, bu_ref, bv_ref, bh_ref, pu_ref, pv_ref):
    bh = bh_ref[...]
    dn = (((0,), (0,)), ((), ()))
    au = wt_ref[...].reshape(EMB_K * K1_NSUB, K1_SUB)
    pu_ref[...] = lax.dot_general(au, bu_ref[...], dn,
                                  preferred_element_type=jnp.float32) + bh
    av = ht_ref[...].reshape(EMB_K * K1_NSUB, K1_SUB)
    pv_ref[...] = lax.dot_general(av, bv_ref[...], dn,
                                  preferred_element_type=jnp.float32) + bh


def _make_tables(WT3, HT3, bu, bv, bh):
    out_shape = jax.ShapeDtypeStruct((NROWS_PACK * PJ // 128, 128),
                                     jnp.float32)
    return pl.pallas_call(
        _tc_body,
        grid=(K1_GRID,),
        in_specs=[
            pl.BlockSpec((EMB_K, K1_NSUB, K1_SUB), lambda g: (0, g, 0)),
            pl.BlockSpec((EMB_K, K1_NSUB, K1_SUB), lambda g: (0, g, 0)),
            pl.BlockSpec((128, 128), lambda g: (0, 0)),
            pl.BlockSpec((128, 128), lambda g: (0, 0)),
            pl.BlockSpec((1, 128), lambda g: (0, 0)),
        ],
        out_specs=[
            pl.BlockSpec((K1_OUT_ROWS, 128), lambda g: (g, 0)),
            pl.BlockSpec((K1_OUT_ROWS, 128), lambda g: (g, 0)),
        ],
        out_shape=[out_shape, out_shape],
    )(WT3, HT3, bu, bv, bh)


def _sc_body(x_hbm, pu_hbm, pv_hbm, out_hbm,
             xv, uidx_v, vidx_v, purows, pvrows, outv, sem):
    wid = lax.axis_index("s") * NUM_CORES + lax.axis_index("c")
    base = wid * BPW

    pltpu.sync_copy(x_hbm.at[pl.ds(base, BPW)], xv)

    iota = lax.iota(jnp.int32, LANES)
    zerov = jnp.zeros((LANES,), jnp.int32)
    onev = jnp.full((LANES,), 1, jnp.int32)
    m_hi = jnp.full((LANES,), -K1_COLS, jnp.int32)
    m_lo = jnp.full((LANES,), K1_SUB - 1, jnp.int32)
    m_mid = jnp.full((LANES,), K1_NSUB - 1, jnp.int32)

    def packed(n):
        return ((n & m_hi) + ((n & m_lo) << 3)) + ((n >> 11) & m_mid)

    for c in range(NCHUNK):
        for o in range(CHUNK // LANES):
            rows = c * CHUNK + o * LANES + iota
            u = plsc.load_gather(xv, [rows, zerov])
            v = plsc.load_gather(xv, [rows, onev])
            uidx_v[c, pl.ds(o * LANES, LANES)] = packed(u)
            vidx_v[c, pl.ds(o * LANES, LANES)] = packed(v)

    copies = []
    for c in range(NCHUNK):
        copies.append(pltpu.async_copy(
            pu_hbm.at[uidx_v.at[c]], purows.at[pl.ds(c * CHUNK, CHUNK)], sem))
        copies.append(pltpu.async_copy(
            pv_hbm.at[vidx_v.at[c]], pvrows.at[pl.ds(c * CHUNK, CHUNK)], sem))
    for cp in copies:
        cp.wait()

    def block(gb, carry):
        row0 = gb * ROWS_PER_BLOCK
        for g in range(GROUPS_PER_BLOCK):
            row_idx = row0 + g * LANES + iota
            h = []
            for j in range(NCLS):
                colv = jnp.full((LANES,), j, jnp.int32)
                pu = plsc.load_gather(purows, [row_idx, colv])
                pv = plsc.load_gather(pvrows, [row_idx, colv])
                h.append(pu + pv)
            m = h[0]
            for j in range(1, NCLS):
                m = jnp.maximum(m, h[j])
            e = [jnp.exp(h[j] - m) for j in range(NCLS)]
            s = e[0]
            for j in range(1, NCLS):
                s = s + e[j]
            r = jnp.full((LANES,), 1.0, jnp.float32) / s
            for j in range(NCLS):
                plsc.store_scatter(
                    outv, [row_idx, jnp.full((LANES,), j, jnp.int32)],
                    e[j] * r)
        return carry

    lax.fori_loop(0, NBLOCKS, block, 0)

    pltpu.sync_copy(outv, out_hbm.at[pl.ds(base, BPW)])


def _score(x, pu, pv):
    mesh = plsc.VectorSubcoreMesh(
        core_axis_name="c", subcore_axis_name="s",
        num_cores=NUM_CORES, num_subcores=NUM_SUBCORES)
    return pl.kernel(
        _sc_body,
        out_type=jax.ShapeDtypeStruct((BATCH, NCLS), jnp.float32),
        mesh=mesh,
        compiler_params=pltpu.CompilerParams(
            needs_layout_passes=False, use_tc_tiling_on_sc=False),
        scratch_types=[
            pltpu.VMEM((BPW, 2), jnp.int32),
            pltpu.VMEM((NCHUNK, CHUNK), jnp.int32),
            pltpu.VMEM((NCHUNK, CHUNK), jnp.int32),
            pltpu.VMEM((BPW, PJ), jnp.float32),
            pltpu.VMEM((BPW, PJ), jnp.float32),
            pltpu.VMEM((BPW, NCLS), jnp.float32),
            pltpu.SemaphoreType.DMA,
        ],
    )(x, pu, pv)


def _blockdiag(w):
    kk = jnp.arange(EMB_K)[:, None, None]
    mm = jnp.arange(K1_NSUB)[None, :, None]
    jj = jnp.arange(NCLS)[None, None, :]
    rows = (K1_NSUB * kk + mm + 0 * jj).reshape(-1)
    cols = (PJ * mm + jj + 0 * kk).reshape(-1)
    vals = jnp.broadcast_to(w.T[:, None, :],
                            (EMB_K, K1_NSUB, NCLS)).reshape(-1)
    return jnp.zeros((128, 128), jnp.float32).at[rows, cols].set(vals)


@jax.jit
def _run(x, W, H, lin_w, lin_b):
    bu = _blockdiag(lin_w[:, :EMB_K])
    bv = _blockdiag(lin_w[:, EMB_K:])
    bh16 = jnp.zeros((1, PJ), jnp.float32).at[0, :NCLS].set(0.5 * lin_b)
    bh = jnp.tile(bh16, (1, K1_NSUB))
    pad = ((0, 0, 0), (0, NROWS_PACK - NROWS, 0))
    WT3 = lax.pad(W.T, jnp.float32(0), pad).reshape(
        EMB_K, NROWS_PACK // K1_SUB, K1_SUB)
    HT3 = lax.pad(H.T, jnp.float32(0), pad).reshape(
        EMB_K, NROWS_PACK // K1_SUB, K1_SUB)
    p2u, p2v = _make_tables(WT3, HT3, bu, bv, bh)
    pu = p2u.reshape(NROWS_PACK, PJ)
    pv = p2v.reshape(NROWS_PACK, PJ)
    return _score(x, pu, pv)


def kernel(x, W, H, lin_w, lin_b):
    return _run(x, W, H, lin_w, lin_b)


# speedup vs baseline: 5.2526x; 2.3445x over previous
import jax
import jax.numpy as jnp
from jax import lax
from jax.experimental import pallas as pl
from jax.experimental.pallas import tpu as pltpu, tpu_sc as plsc

NUM_CORES = 2
NUM_SUBCORES = 16
LANES = 16
NW = NUM_CORES * NUM_SUBCORES

BATCH = 16384
EMB_K = 16
NCLS = 5
PJ = 16
NROWS = 1000000
NROWS_PAD = 1 << 20
BPW = BATCH // NW
CHUNK = 128
NCHUNK = BPW // CHUNK
GROUPS_PER_BLOCK = 4
ROWS_PER_BLOCK = GROUPS_PER_BLOCK * LANES
NBLOCKS = BPW // ROWS_PER_BLOCK

K1_COLS = 16384
K1_GRID = -(-NROWS // K1_COLS)
K1_NSUB = 128 // PJ
K1_SUB = K1_COLS // K1_NSUB
K1_OUT_ROWS = K1_SUB
NROWS_PACK = K1_GRID * K1_COLS


def _tc_body(wt_ref, ht_ref, bu_ref, bv_ref, bh_ref, pu_ref, pv_ref):
    bh = bh_ref[...]
    dn = (((0,), (0,)), ((), ()))
    au = jnp.concatenate(
        [wt_ref[:, pl.ds(m * K1_SUB, K1_SUB)] for m in range(K1_NSUB)], 0)
    pu_ref[...] = lax.dot_general(au, bu_ref[...], dn,
                                  preferred_element_type=jnp.float32) + bh
    av = jnp.concatenate(
        [ht_ref[:, pl.ds(m * K1_SUB, K1_SUB)] for m in range(K1_NSUB)], 0)
    pv_ref[...] = lax.dot_general(av, bv_ref[...], dn,
                                  preferred_element_type=jnp.float32) + bh


def _make_tables(WT3, HT3, bu, bv, bh):
    out_shape = jax.ShapeDtypeStruct((NROWS_PACK * PJ // 128, 128),
                                     jnp.float32)
    return pl.pallas_call(
        _tc_body,
        grid=(K1_GRID,),
        in_specs=[
            pl.BlockSpec((EMB_K, K1_COLS), lambda g: (0, g)),
            pl.BlockSpec((EMB_K, K1_COLS), lambda g: (0, g)),
            pl.BlockSpec((128, 128), lambda g: (0, 0)),
            pl.BlockSpec((128, 128), lambda g: (0, 0)),
            pl.BlockSpec((1, 128), lambda g: (0, 0)),
        ],
        out_specs=[
            pl.BlockSpec((K1_OUT_ROWS, 128), lambda g: (g, 0)),
            pl.BlockSpec((K1_OUT_ROWS, 128), lambda g: (g, 0)),
        ],
        out_shape=[out_shape, out_shape],
    )(WT3, HT3, bu, bv, bh)


def _sc_body(x_hbm, pu_hbm, pv_hbm, out_hbm,
             xv, uidx_v, vidx_v, purows, pvrows, outv, sem):
    wid = lax.axis_index("s") * NUM_CORES + lax.axis_index("c")
    base = wid * BPW

    pltpu.sync_copy(x_hbm.at[pl.ds(base, BPW)], xv)

    iota = lax.iota(jnp.int32, LANES)
    zerov = jnp.zeros((LANES,), jnp.int32)
    onev = jnp.full((LANES,), 1, jnp.int32)
    m_hi = jnp.full((LANES,), -K1_COLS, jnp.int32)
    m_lo = jnp.full((LANES,), K1_SUB - 1, jnp.int32)
    m_mid = jnp.full((LANES,), K1_NSUB - 1, jnp.int32)

    def packed(n):
        return ((n & m_hi) + ((n & m_lo) << 3)) + ((n >> 11) & m_mid)

    for c in range(NCHUNK):
        for o in range(CHUNK // LANES):
            rows = c * CHUNK + o * LANES + iota
            u = plsc.load_gather(xv, [rows, zerov])
            v = plsc.load_gather(xv, [rows, onev])
            uidx_v[c, pl.ds(o * LANES, LANES)] = packed(u)
            vidx_v[c, pl.ds(o * LANES, LANES)] = packed(v)

    copies = []
    for c in range(NCHUNK):
        copies.append(pltpu.async_copy(
            pu_hbm.at[uidx_v.at[c]], purows.at[pl.ds(c * CHUNK, CHUNK)], sem))
        copies.append(pltpu.async_copy(
            pv_hbm.at[vidx_v.at[c]], pvrows.at[pl.ds(c * CHUNK, CHUNK)], sem))
    for cp in copies:
        cp.wait()

    def block(gb, carry):
        row0 = gb * ROWS_PER_BLOCK
        for g in range(GROUPS_PER_BLOCK):
            row_idx = row0 + g * LANES + iota
            h = []
            for j in range(NCLS):
                colv = jnp.full((LANES,), j, jnp.int32)
                pu = plsc.load_gather(purows, [row_idx, colv])
                pv = plsc.load_gather(pvrows, [row_idx, colv])
                h.append(pu + pv)
            m = h[0]
            for j in range(1, NCLS):
                m = jnp.maximum(m, h[j])
            e = [jnp.exp(h[j] - m) for j in range(NCLS)]
            s = e[0]
            for j in range(1, NCLS):
                s = s + e[j]
            r = jnp.full((LANES,), 1.0, jnp.float32) / s
            for j in range(NCLS):
                plsc.store_scatter(
                    outv, [row_idx, jnp.full((LANES,), j, jnp.int32)],
                    e[j] * r)
        return carry

    lax.fori_loop(0, NBLOCKS, block, 0)

    pltpu.sync_copy(outv, out_hbm.at[pl.ds(base, BPW)])


def _score(x, pu, pv):
    mesh = plsc.VectorSubcoreMesh(
        core_axis_name="c", subcore_axis_name="s",
        num_cores=NUM_CORES, num_subcores=NUM_SUBCORES)
    return pl.kernel(
        _sc_body,
        out_type=jax.ShapeDtypeStruct((BATCH, NCLS), jnp.float32),
        mesh=mesh,
        compiler_params=pltpu.CompilerParams(
            needs_layout_passes=False, use_tc_tiling_on_sc=False),
        scratch_types=[
            pltpu.VMEM((BPW, 2), jnp.int32),
            pltpu.VMEM((NCHUNK, CHUNK), jnp.int32),
            pltpu.VMEM((NCHUNK, CHUNK), jnp.int32),
            pltpu.VMEM((BPW, PJ), jnp.float32),
            pltpu.VMEM((BPW, PJ), jnp.float32),
            pltpu.VMEM((BPW, NCLS), jnp.float32),
            pltpu.SemaphoreType.DMA,
        ],
    )(x, pu, pv)


def _blockdiag(w):
    wt = jnp.zeros((EMB_K, PJ), jnp.float32).at[:, :NCLS].set(w.T)
    eye = jnp.eye(K1_NSUB, dtype=jnp.float32)
    return (eye[:, None, :, None] * wt[None, :, None, :]).reshape(128, 128)


@jax.jit
def _run(x, W, H, lin_w, lin_b):
    bu = _blockdiag(lin_w[:, :EMB_K])
    bv = _blockdiag(lin_w[:, EMB_K:])
    bh16 = jnp.zeros((1, PJ), jnp.float32).at[0, :NCLS].set(0.5 * lin_b)
    bh = jnp.tile(bh16, (1, K1_NSUB))
    p2u, p2v = _make_tables(W.T, H.T, bu, bv, bh)
    pu = p2u.reshape(NROWS_PACK, PJ)
    pv = p2v.reshape(NROWS_PACK, PJ)
    return _score(x, pu, pv)


def kernel(x, W, H, lin_w, lin_b):
    return _run(x, W, H, lin_w, lin_b)


# speedup vs baseline: 5.7941x; 1.1031x over previous
import jax
import jax.numpy as jnp
from jax import lax
from jax.experimental import pallas as pl
from jax.experimental.pallas import tpu as pltpu, tpu_sc as plsc

NUM_CORES = 2
NUM_SUBCORES = 16
LANES = 16
NW = NUM_CORES * NUM_SUBCORES

BATCH = 16384
EMB_K = 16
NCLS = 5
PJ = 16
NROWS = 1000000
NROWS_PAD = 1 << 20
BPW = BATCH // NW
CHUNK = 128
NCHUNK = BPW // CHUNK
GROUPS_PER_BLOCK = 4
ROWS_PER_BLOCK = GROUPS_PER_BLOCK * LANES
NBLOCKS = BPW // ROWS_PER_BLOCK

K1_COLS = 32768
K1_GRID = -(-NROWS // K1_COLS)
K1_NSUB = 128 // PJ
K1_SUB = K1_COLS // K1_NSUB
K1_SUB_LOG = K1_SUB.bit_length() - 1
K1_OUT_ROWS = K1_SUB
NROWS_PACK = K1_GRID * K1_COLS


def _tc_body(wt_ref, ht_ref, bu_ref, bv_ref, bh_ref, pu_ref, pv_ref):
    bh = bh_ref[...]
    dn = (((0,), (0,)), ((), ()))
    base = pl.program_id(0) * K1_COLS
    iota_l = lax.broadcasted_iota(jnp.int32, (EMB_K, K1_SUB), 1)

    def fold(ref):
        return jnp.concatenate(
            [jnp.where(base + (m * K1_SUB) + iota_l < NROWS,
                       ref[:, pl.ds(m * K1_SUB, K1_SUB)], 0.0)
             for m in range(K1_NSUB)], 0)

    pu_ref[...] = lax.dot_general(fold(wt_ref), bu_ref[...], dn,
                                  preferred_element_type=jnp.float32) + bh
    pv_ref[...] = lax.dot_general(fold(ht_ref), bv_ref[...], dn,
                                  preferred_element_type=jnp.float32) + bh


def _make_tables(WT3, HT3, bu, bv, bh):
    out_shape = jax.ShapeDtypeStruct((NROWS_PACK * PJ // 128, 128),
                                     jnp.float32)
    return pl.pallas_call(
        _tc_body,
        grid=(K1_GRID,),
        in_specs=[
            pl.BlockSpec((EMB_K, K1_COLS), lambda g: (0, g)),
            pl.BlockSpec((EMB_K, K1_COLS), lambda g: (0, g)),
            pl.BlockSpec((128, 128), lambda g: (0, 0)),
            pl.BlockSpec((128, 128), lambda g: (0, 0)),
            pl.BlockSpec((1, 128), lambda g: (0, 0)),
        ],
        out_specs=[
            pl.BlockSpec((K1_OUT_ROWS, 128), lambda g: (g, 0)),
            pl.BlockSpec((K1_OUT_ROWS, 128), lambda g: (g, 0)),
        ],
        out_shape=[out_shape, out_shape],
    )(WT3, HT3, bu, bv, bh)


def _sc_body(x_hbm, pu_hbm, pv_hbm, out_hbm,
             xv, uidx_v, vidx_v, purows, pvrows, outv, sem):
    wid = lax.axis_index("s") * NUM_CORES + lax.axis_index("c")
    base = wid * BPW

    pltpu.sync_copy(x_hbm.at[pl.ds(base, BPW)], xv)

    iota = lax.iota(jnp.int32, LANES)
    zerov = jnp.zeros((LANES,), jnp.int32)
    onev = jnp.full((LANES,), 1, jnp.int32)
    m_hi = jnp.full((LANES,), -K1_COLS, jnp.int32)
    m_lo = jnp.full((LANES,), K1_SUB - 1, jnp.int32)
    m_mid = jnp.full((LANES,), K1_NSUB - 1, jnp.int32)

    def packed(n):
        return ((n & m_hi) + ((n & m_lo) << 3)) + ((n >> K1_SUB_LOG) & m_mid)

    for c in range(NCHUNK):
        for o in range(CHUNK // LANES):
            rows = c * CHUNK + o * LANES + iota
            u = plsc.load_gather(xv, [rows, zerov])
            v = plsc.load_gather(xv, [rows, onev])
            uidx_v[c, pl.ds(o * LANES, LANES)] = packed(u)
            vidx_v[c, pl.ds(o * LANES, LANES)] = packed(v)

    copies = []
    for c in range(NCHUNK):
        copies.append(pltpu.async_copy(
            pu_hbm.at[uidx_v.at[c]], purows.at[pl.ds(c * CHUNK, CHUNK)], sem))
        copies.append(pltpu.async_copy(
            pv_hbm.at[vidx_v.at[c]], pvrows.at[pl.ds(c * CHUNK, CHUNK)], sem))
    for cp in copies:
        cp.wait()

    def block(gb, carry):
        row0 = gb * ROWS_PER_BLOCK
        for g in range(GROUPS_PER_BLOCK):
            row_idx = row0 + g * LANES + iota
            h = []
            for j in range(NCLS):
                colv = jnp.full((LANES,), j, jnp.int32)
                pu = plsc.load_gather(purows, [row_idx, colv])
                pv = plsc.load_gather(pvrows, [row_idx, colv])
                h.append(pu + pv)
            m = h[0]
            for j in range(1, NCLS):
                m = jnp.maximum(m, h[j])
            e = [jnp.exp(h[j] - m) for j in range(NCLS)]
            s = e[0]
            for j in range(1, NCLS):
                s = s + e[j]
            r = jnp.full((LANES,), 1.0, jnp.float32) / s
            for j in range(NCLS):
                plsc.store_scatter(
                    outv, [row_idx, jnp.full((LANES,), j, jnp.int32)],
                    e[j] * r)
        return carry

    lax.fori_loop(0, NBLOCKS, block, 0)

    pltpu.sync_copy(outv, out_hbm.at[pl.ds(base, BPW)])


def _score(x, pu, pv):
    mesh = plsc.VectorSubcoreMesh(
        core_axis_name="c", subcore_axis_name="s",
        num_cores=NUM_CORES, num_subcores=NUM_SUBCORES)
    return pl.kernel(
        _sc_body,
        out_type=jax.ShapeDtypeStruct((BATCH, NCLS), jnp.float32),
        mesh=mesh,
        compiler_params=pltpu.CompilerParams(
            needs_layout_passes=False, use_tc_tiling_on_sc=False),
        scratch_types=[
            pltpu.VMEM((BPW, 2), jnp.int32),
            pltpu.VMEM((NCHUNK, CHUNK), jnp.int32),
            pltpu.VMEM((NCHUNK, CHUNK), jnp.int32),
            pltpu.VMEM((BPW, PJ), jnp.float32),
            pltpu.VMEM((BPW, PJ), jnp.float32),
            pltpu.VMEM((BPW, NCLS), jnp.float32),
            pltpu.SemaphoreType.DMA,
        ],
    )(x, pu, pv)


def _blockdiag(w):
    wt = jnp.zeros((EMB_K, PJ), jnp.float32).at[:, :NCLS].set(w.T)
    eye = jnp.eye(K1_NSUB, dtype=jnp.float32)
    return (eye[:, None, :, None] * wt[None, :, None, :]).reshape(128, 128)


@jax.jit
def _run(x, W, H, lin_w, lin_b):
    bu = _blockdiag(lin_w[:, :EMB_K])
    bv = _blockdiag(lin_w[:, EMB_K:])
    bh16 = jnp.zeros((1, PJ), jnp.float32).at[0, :NCLS].set(0.5 * lin_b)
    bh = jnp.tile(bh16, (1, K1_NSUB))
    p2u, p2v = _make_tables(W.T, H.T, bu, bv, bh)
    pu = p2u.reshape(NROWS_PACK, PJ)
    pv = p2v.reshape(NROWS_PACK, PJ)
    return _score(x, pu, pv)


def kernel(x, W, H, lin_w, lin_b):
    return _run(x, W, H, lin_w, lin_b)
